# baseline (speedup 1.0000x reference)
import functools

import jax
import jax.numpy as jnp
from jax import lax
from jax.experimental import pallas as pl
from jax.experimental.pallas import tpu as pltpu
from jax.experimental.pallas import tpu_sc as plsc

B = 16384
F = 26
FIELD = 40000
L = 16
NC = 2
NS = 16
NW = NC * NS
BPW = B // NW
C = 8
RPC = BPW // C
GPC = RPC // L
CHUNK = F * RPC


def _sc_embed_sum(x, fc_flat, bias16):
    mesh = plsc.VectorSubcoreMesh(core_axis_name="c", subcore_axis_name="s")

    @functools.partial(
        pl.kernel,
        out_type=jax.ShapeDtypeStruct((B,), jnp.float32),
        mesh=mesh,
        scratch_types=[
            pltpu.VMEM((F * BPW,), jnp.int32),
            pltpu.VMEM((F * BPW,), jnp.float32),
            pltpu.VMEM((L,), jnp.float32),
            pltpu.VMEM((BPW,), jnp.float32),
        ] + [pltpu.SemaphoreType.DMA] * C,
    )
    def k(x_hbm, fc_hbm, bias_hbm, out_hbm, idxv, valv, biasv, outv, *sems):
        wid = lax.axis_index("s") * NC + lax.axis_index("c")
        base = wid * BPW
        xcopies = [
            pltpu.async_copy(x_hbm.at[wid, pl.ds(c * CHUNK, CHUNK)],
                             idxv.at[pl.ds(c * CHUNK, CHUNK)], sems[c])
            for c in range(C)
        ]
        pltpu.sync_copy(bias_hbm, biasv)

        gcopies = []
        for c in range(C):
            xcopies[c].wait()
            cb = c * CHUNK
            for f in range(F):
                off = f * FIELD
                for j in range(GPC):
                    s = pl.ds(cb + f * RPC + j * L, L)
                    idxv[s] = idxv[s] + off
            cs = pl.ds(cb, CHUNK)
            gcopies.append(
                pltpu.async_copy(fc_hbm.at[idxv.at[cs]], valv.at[cs], sems[c])
            )

        bias_vec = biasv[...]
        ocopies = []
        for c in range(C):
            gcopies[c].wait()
            cb = c * CHUNK
            for j in range(GPC):
                acc = valv[pl.ds(cb + j * L, L)] + bias_vec
                for f in range(1, F):
                    acc = acc + valv[pl.ds(cb + f * RPC + j * L, L)]
                outv[pl.ds(c * RPC + j * L, L)] = acc
            ocopies.append(
                pltpu.async_copy(outv.at[pl.ds(c * RPC, RPC)],
                                 out_hbm.at[pl.ds(base + c * RPC, RPC)],
                                 sems[c])
            )
        for c in range(C):
            ocopies[c].wait()

    return k(x, fc_flat, bias16)


def kernel(x, fc, bias):
    xa = x.reshape(NW, C, RPC, F).transpose(0, 1, 3, 2).reshape(NW, F * BPW)
    fc_flat = fc.reshape(-1)
    bias16 = jnp.broadcast_to(bias, (L,))
    out = _sc_embed_sum(xa, fc_flat, bias16)
    return out.reshape(B, 1)


# speedup vs baseline: 1.0356x; 1.0356x over previous
import functools

import jax
import jax.numpy as jnp
from jax import lax
from jax.experimental import pallas as pl
from jax.experimental.pallas import tpu as pltpu
from jax.experimental.pallas import tpu_sc as plsc

B = 16384
F = 26
FIELD = 40000
L = 16
NC = 2
NS = 16
NW = NC * NS
BPW = B // NW
C = 2
RPC = BPW // C
GPC = RPC // L
CHUNK = F * RPC


def _sc_embed_sum(x, fc_flat, bias16):
    mesh = plsc.VectorSubcoreMesh(core_axis_name="c", subcore_axis_name="s")

    @functools.partial(
        pl.kernel,
        out_type=jax.ShapeDtypeStruct((B,), jnp.float32),
        mesh=mesh,
        scratch_types=[
            pltpu.VMEM((F * BPW,), jnp.int32),
            pltpu.VMEM((F * BPW,), jnp.float32),
            pltpu.VMEM((L,), jnp.float32),
            pltpu.VMEM((BPW,), jnp.float32),
        ] + [pltpu.SemaphoreType.DMA] * C,
    )
    def k(x_hbm, fc_hbm, bias_hbm, out_hbm, idxv, valv, biasv, outv, *sems):
        wid = lax.axis_index("s") * NC + lax.axis_index("c")
        base = wid * BPW
        xcopies = [
            pltpu.async_copy(x_hbm.at[wid, pl.ds(c * CHUNK, CHUNK)],
                             idxv.at[pl.ds(c * CHUNK, CHUNK)], sems[c])
            for c in range(C)
        ]
        pltpu.sync_copy(bias_hbm, biasv)

        gcopies = []
        for c in range(C):
            xcopies[c].wait()
            cb = c * CHUNK
            for f in range(F):
                off = f * FIELD
                for j in range(GPC):
                    s = pl.ds(cb + f * RPC + j * L, L)
                    idxv[s] = idxv[s] + off
            cs = pl.ds(cb, CHUNK)
            gcopies.append(
                pltpu.async_copy(fc_hbm.at[idxv.at[cs]], valv.at[cs], sems[c])
            )

        bias_vec = biasv[...]
        ocopies = []
        for c in range(C):
            gcopies[c].wait()
            cb = c * CHUNK
            for j in range(GPC):
                acc = valv[pl.ds(cb + j * L, L)] + bias_vec
                for f in range(1, F):
                    acc = acc + valv[pl.ds(cb + f * RPC + j * L, L)]
                outv[pl.ds(c * RPC + j * L, L)] = acc
            ocopies.append(
                pltpu.async_copy(outv.at[pl.ds(c * RPC, RPC)],
                                 out_hbm.at[pl.ds(base + c * RPC, RPC)],
                                 sems[c])
            )
        for c in range(C):
            ocopies[c].wait()

    return k(x, fc_flat, bias16)


def kernel(x, fc, bias):
    xa = x.reshape(NW, C, RPC, F).transpose(0, 1, 3, 2).reshape(NW, F * BPW)
    fc_flat = fc.reshape(-1)
    bias16 = jnp.broadcast_to(bias, (L,))
    out = _sc_embed_sum(xa, fc_flat, bias16)
    return out.reshape(B, 1)


# speedup vs baseline: 1.0583x; 1.0219x over previous
import functools

import jax
import jax.numpy as jnp
from jax import lax
from jax.experimental import pallas as pl
from jax.experimental.pallas import tpu as pltpu
from jax.experimental.pallas import tpu_sc as plsc

B = 16384
F = 26
FIELD = 40000
L = 16
NC = 2
NS = 16
NW = NC * NS
BPW = B // NW
C = 4
RPC = BPW // C
GPC = RPC // L
CHUNK = F * RPC


def _sc_embed_sum(x, fc_flat, bias16):
    mesh = plsc.VectorSubcoreMesh(core_axis_name="c", subcore_axis_name="s")

    @functools.partial(
        pl.kernel,
        out_type=jax.ShapeDtypeStruct((B,), jnp.float32),
        mesh=mesh,
        scratch_types=[
            pltpu.VMEM((F * BPW,), jnp.int32),
            pltpu.VMEM((F * BPW,), jnp.float32),
            pltpu.VMEM((L,), jnp.float32),
            pltpu.VMEM((BPW,), jnp.float32),
        ] + [pltpu.SemaphoreType.DMA] * C,
    )
    def k(x_hbm, fc_hbm, bias_hbm, out_hbm, idxv, valv, biasv, outv, *sems):
        wid = lax.axis_index("s") * NC + lax.axis_index("c")
        base = wid * BPW
        xcopies = [
            pltpu.async_copy(x_hbm.at[wid, pl.ds(c * CHUNK, CHUNK)],
                             idxv.at[pl.ds(c * CHUNK, CHUNK)], sems[c])
            for c in range(C)
        ]
        pltpu.sync_copy(bias_hbm, biasv)

        gcopies = []
        for c in range(C):
            xcopies[c].wait()
            cb = c * CHUNK
            cs = pl.ds(cb, CHUNK)
            gcopies.append(
                pltpu.async_copy(fc_hbm.at[idxv.at[cs]], valv.at[cs], sems[c])
            )

        bias_vec = biasv[...]
        ocopies = []
        for c in range(C):
            gcopies[c].wait()
            cb = c * CHUNK
            for j in range(GPC):
                acc = valv[pl.ds(cb + j * L, L)] + bias_vec
                for f in range(1, F):
                    acc = acc + valv[pl.ds(cb + f * RPC + j * L, L)]
                outv[pl.ds(c * RPC + j * L, L)] = acc
            ocopies.append(
                pltpu.async_copy(outv.at[pl.ds(c * RPC, RPC)],
                                 out_hbm.at[pl.ds(base + c * RPC, RPC)],
                                 sems[c])
            )
        for c in range(C):
            ocopies[c].wait()

    return k(x, fc_flat, bias16)


def kernel(x, fc, bias):
    offs = jnp.arange(F, dtype=jnp.int32) * FIELD
    xa = (x + offs[None, :]).reshape(NW, C, RPC, F).transpose(0, 1, 3, 2)
    xa = xa.reshape(NW, F * BPW)
    fc_flat = fc.reshape(-1)
    bias16 = jnp.broadcast_to(bias, (L,))
    out = _sc_embed_sum(xa, fc_flat, bias16)
    return out.reshape(B, 1)
